# Initial kernel scaffold; baseline (speedup 1.0000x reference)
import jax
import jax.numpy as jnp
from jax.experimental import pallas as pl


def kernel(x, source, W_group, r):
    raise NotImplementedError("write your pallas kernel here")


# speedup vs baseline: 26.0990x; 26.0990x over previous
import functools

import jax
import jax.numpy as jnp
from jax.experimental import pallas as pl

W = 16
NW1 = 64
WB3 = 32


def _stage1_body(x_ref, wg_ref, wx_ref, gcol_ref, *, rp, dq):
    nw = NW1
    xb = x_ref[...]
    wg = wg_ref[...]
    m = jax.lax.dot_general(xb, wg, (((1,), (1,)), ((), ())),
                            preferred_element_type=jnp.float32)
    nrm = jnp.sqrt(jnp.sum(m * m, axis=1, keepdims=True))
    m = m / jnp.maximum(nrm, 1e-12)
    t1 = W // 2
    mr = m.reshape(nw * t1, 2, dq)
    a = mr[:, 0, :]
    b = mr[:, 1, :]
    aw = a.reshape(nw, t1, dq)
    bw = b.reshape(nw, t1, dq)
    sc = jax.lax.dot_general(aw, bw, (((2,), (2,)), ((0,), (0,))),
                             preferred_element_type=jnp.float32)
    nm = jnp.max(sc, axis=2)
    jj3 = jax.lax.broadcasted_iota(jnp.int32, (nw, t1, t1), 2)
    ismax = sc >= nm[:, :, None]
    nidx = jnp.min(jnp.where(ismax, jj3, t1), axis=2)
    nmi = nm[:, :, None]
    nmj = nm[:, None, :]
    ii3 = jax.lax.broadcasted_iota(jnp.int32, (nw, t1, t1), 1)
    beats = (nmj > nmi) | ((nmj == nmi) & (jj3 < ii3))
    rank = jnp.sum(beats.astype(jnp.int32), axis=2)
    pcount = jnp.sum(((rank[:, None, :] < rank[:, :, None])
                      & (rank[:, None, :] < rp)
                      & (nidx[:, None, :] == nidx[:, :, None])
                      ).astype(jnp.int32), axis=2)

    ns = W - rp
    rk = rank[:, None, :]
    nx = nidx[:, None, :]
    pc = pcount[:, None, :]
    s3 = jax.lax.broadcasted_iota(jnp.int32, (nw, ns, t1), 1)
    tok = jax.lax.broadcasted_iota(jnp.int32, (nw, ns, t1), 2)
    unm_col = jnp.sum(jnp.where(rk == s3 + rp, 2 * tok, 0), axis=2)
    s2 = jax.lax.broadcasted_iota(jnp.int32, (nw, ns), 1)
    prim = jnp.where(s2 < rp, unm_col, 2 * (s2 - rp) + 1)
    cols = [prim]
    for p in range(1, rp + 1):
        match = (rk < rp) & (nx == s3 - rp) & (pc == p - 1)
        mcol = jnp.sum(jnp.where(match, 2 * tok, 0), axis=2)
        has = jnp.sum(match.astype(jnp.int32), axis=2)
        cols.append(jnp.where(has > 0, mcol, prim))
    size = jnp.full((nw, ns), 1.0, jnp.float32)
    for p in range(1, rp + 1):
        size = size + (cols[p] != prim).astype(jnp.float32)

    k3 = jax.lax.broadcasted_iota(jnp.int32, (nw, ns, W), 2)
    s33 = jax.lax.broadcasted_iota(jnp.int32, (nw, ns, W), 1)
    ad = (k3 == prim[:, :, None]).astype(jnp.float32)
    for p in range(1, rp + 1):
        ad = ad + (((cols[p] != prim)[:, :, None])
                   & (k3 == cols[p][:, :, None])).astype(jnp.float32)
    aun = (k3 == unm_col[:, :, None]).astype(jnp.float32)
    wx = jnp.where(s33 < rp, aun, ad / size[:, :, None])
    wx_ref[...] = wx

    woff = (jax.lax.broadcasted_iota(jnp.int32, (nw, ns), 0) % 8) * W
    stacked = []
    for p in range(rp + 1):
        g3 = (cols[p] + woff).reshape(nw // 8, 8, ns)
        g = jnp.concatenate([g3[:, k, :] for k in range(8)], axis=1)
        stacked.append(g)
    gcol_ref[...] = jnp.stack(stacked, axis=1)


def _stage2_body(wx_ref, x_ref, o_ref, *, ns):
    xb = x_ref[...]
    wxr = wx_ref[...].reshape(8 * ns, W)
    at = jnp.concatenate([wxr] * 8, axis=1)
    ri = jax.lax.broadcasted_iota(jnp.int32, (8 * ns, 8 * W), 0) // ns
    ci = jax.lax.broadcasted_iota(jnp.int32, (8 * ns, 8 * W), 1) // W
    A = jnp.where(ri == ci, at, 0.0)
    o_ref[...] = jax.lax.dot_general(
        A, xb, (((1,), (0,)), ((), ())),
        preferred_element_type=jnp.float32,
        precision=jax.lax.Precision.HIGHEST)


def _stage3_body(gcol_ref, src_ref, o_ref, *, rp, ns):
    sb = src_ref[0]
    g = gcol_ref[...]
    nblk = WB3 // 8
    ci = jax.lax.broadcasted_iota(jnp.int32, (WB3 * W, WB3 * ns), 0)
    acc = None
    for p in range(rp + 1):
        tgt = jnp.concatenate(
            [g[q, p:p + 1, :] + q * 8 * W for q in range(nblk)], axis=1)
        G = (ci == tgt).astype(jnp.float32)
        gath = jax.lax.dot_general(sb, G, (((1,), (0,)), ((), ())),
                                   preferred_element_type=jnp.float32)
        acc = gath if acc is None else jnp.maximum(acc, gath)
    o_ref[0] = acc


def kernel(x, source, W_group, r):
    B, N, D = x.shape
    dq = W_group.shape[0]
    rp = min(D // dq, W // 2)
    ns = W - rp
    num_windows = N // W
    tw = B * num_windows

    x2 = x.reshape(B * N, D)

    wx, gcol = pl.pallas_call(
        functools.partial(_stage1_body, rp=rp, dq=dq),
        grid=(tw // NW1,),
        in_specs=[
            pl.BlockSpec((NW1 * W, D), lambda i: (i, 0)),
            pl.BlockSpec((dq, D), lambda i: (0, 0)),
        ],
        out_specs=[
            pl.BlockSpec((NW1, ns, W), lambda i: (i, 0, 0)),
            pl.BlockSpec((NW1 // 8, rp + 1, 8 * ns), lambda i: (i, 0, 0)),
        ],
        out_shape=[
            jax.ShapeDtypeStruct((tw, ns, W), jnp.float32),
            jax.ShapeDtypeStruct((tw // 8, rp + 1, 8 * ns), jnp.int32),
        ],
    )(x2, W_group)

    xm2 = pl.pallas_call(
        functools.partial(_stage2_body, ns=ns),
        grid=(tw // 8,),
        in_specs=[
            pl.BlockSpec((8, ns, W), lambda i: (i, 0, 0)),
            pl.BlockSpec((8 * W, D), lambda i: (i, 0)),
        ],
        out_specs=pl.BlockSpec((8 * ns, D), lambda i: (i, 0)),
        out_shape=jax.ShapeDtypeStruct((tw * ns, D), jnp.float32),
    )(wx, x2)
    x_merged = xm2.reshape(B, num_windows * ns, D)

    n_rows = source.shape[1]
    s_m = pl.pallas_call(
        functools.partial(_stage3_body, rp=rp, ns=ns),
        grid=(B, num_windows // WB3),
        in_specs=[
            pl.BlockSpec((WB3 // 8, rp + 1, 8 * ns),
                         lambda b, wb: (b * (num_windows // WB3) + wb, 0, 0)),
            pl.BlockSpec((1, n_rows, WB3 * W), lambda b, wb: (b, 0, wb)),
        ],
        out_specs=pl.BlockSpec((1, n_rows, WB3 * ns), lambda b, wb: (b, 0, wb)),
        out_shape=jax.ShapeDtypeStruct((B, n_rows, num_windows * ns), jnp.float32),
    )(gcol, source)

    return (x_merged, s_m)


# speedup vs baseline: 64.1773x; 2.4590x over previous
import functools

import jax
import jax.numpy as jnp
from jax.experimental import pallas as pl

W = 16
NW1 = 64
WB3 = 32


def _stage1_body(x_ref, wg_ref, xm_ref, scol_ref, pcol_ref, *, rp, dq):
    nw = NW1
    xb = x_ref[...]
    wg = wg_ref[...]
    m = jax.lax.dot_general(xb, wg, (((1,), (1,)), ((), ())),
                            preferred_element_type=jnp.float32)
    nrm = jnp.sqrt(jnp.sum(m * m, axis=1, keepdims=True))
    m = m / jnp.maximum(nrm, 1e-12)
    t1 = W // 2
    mr = m.reshape(nw * t1, 2, dq)
    a = mr[:, 0, :]
    b = mr[:, 1, :]
    aw = a.reshape(nw, t1, dq)
    bw = b.reshape(nw, t1, dq)
    sc = jax.lax.dot_general(aw, bw, (((2,), (2,)), ((0,), (0,))),
                             preferred_element_type=jnp.float32)
    nm = jnp.max(sc, axis=2)
    jj3 = jax.lax.broadcasted_iota(jnp.int32, (nw, t1, t1), 2)
    ismax = sc >= nm[:, :, None]
    nidx = jnp.min(jnp.where(ismax, jj3, t1), axis=2)
    ii3 = jax.lax.broadcasted_iota(jnp.int32, (nw, t1, t1), 1)
    beats = (nm[:, None, :] > nm[:, :, None]) | (
        (nm[:, None, :] == nm[:, :, None]) & (jj3 < ii3))
    rank = jnp.sum(beats.astype(jnp.int32), axis=2)
    pcount = jnp.sum(((rank[:, None, :] < rank[:, :, None])
                      & (rank[:, None, :] < rp)
                      & (nidx[:, None, :] == nidx[:, :, None])
                      ).astype(jnp.int32), axis=2)

    ns = W - rp
    is_src = rank < rp
    slot_e = jnp.where(is_src, rp + nidx, rank - rp)
    pass_e = jnp.where(is_src, 1 + pcount, 0)
    dj3 = jax.lax.broadcasted_iota(jnp.int32, (nw, t1, t1), 1)
    cnt = jnp.sum((is_src[:, None, :] & (nidx[:, None, :] == dj3)
                   ).astype(jnp.float32), axis=2)
    szd = 1.0 + cnt
    sz_at = jnp.take_along_axis(szd, nidx, axis=1)
    wt_e = jnp.where(is_src, 1.0 / sz_at, 1.0)
    wt_d = 1.0 / szd

    k2 = jax.lax.broadcasted_iota(jnp.int32, (nw, W), 1)
    gidx = jnp.where(k2 % 2 == 0, k2 // 2, t1 + k2 // 2)
    d8 = jax.lax.broadcasted_iota(jnp.int32, (nw, t1), 1)
    slot_cat = jnp.concatenate([slot_e, rp + d8], axis=1)
    pass_cat = jnp.concatenate([pass_e, jnp.zeros_like(pass_e)], axis=1)
    wt_cat = jnp.concatenate([wt_e, wt_d], axis=1)
    slotk = jnp.take_along_axis(slot_cat, gidx, axis=1)
    passk = jnp.take_along_axis(pass_cat, gidx, axis=1)
    wtk = jnp.take_along_axis(wt_cat, gidx, axis=1)

    s3b = jax.lax.broadcasted_iota(jnp.int32, (nw, ns, W), 1)
    wx = (s3b == slotk[:, None, :]).astype(jnp.float32) * wtk[:, None, :]
    ri = jax.lax.broadcasted_iota(jnp.int32, (8 * ns, 8 * W), 0) // ns
    ci = jax.lax.broadcasted_iota(jnp.int32, (8 * ns, 8 * W), 1) // W
    bd = ri == ci
    for g in range(nw // 8):
        wxr = wx[8 * g:8 * g + 8].reshape(8 * ns, W)
        A = jnp.where(bd, jnp.concatenate([wxr] * 8, axis=1), 0.0)
        xg = xb[8 * W * g:8 * W * (g + 1), :]
        xm_ref[8 * ns * g:8 * ns * (g + 1), :] = jax.lax.dot_general(
            A, xg, (((1,), (0,)), ((), ())), preferred_element_type=jnp.float32)

    woff = (jax.lax.broadcasted_iota(jnp.int32, (nw, W), 0) % 8) * ns
    sfull = (slotk + woff).reshape(nw // 8, 8, W)
    pfull = passk.reshape(nw // 8, 8, W)
    scol_ref[...] = jnp.concatenate(
        [sfull[:, k, :] for k in range(8)], axis=1).reshape(nw // 8, 1, 8 * W)
    pcol_ref[...] = jnp.concatenate(
        [pfull[:, k, :] for k in range(8)], axis=1).reshape(nw // 8, 1, 8 * W)


def _stage3_body(scol_ref, pcol_ref, src_ref, o_ref, *, rp, ns):
    sb = src_ref[0]
    sc_ = scol_ref[...]
    pc_ = pcol_ref[...]
    d2 = jax.lax.broadcasted_iota(jnp.int32, (16 * ns, 16 * W), 0)
    accs = []
    for h in range(WB3 // 16):
        srow = jnp.concatenate(
            [sc_[2 * h, :, :], sc_[2 * h + 1, :, :] + 8 * ns],
            axis=1)
        prow = jnp.concatenate(
            [pc_[2 * h, :, :], pc_[2 * h + 1, :, :]], axis=1)
        sbh = sb[:, 16 * W * h:16 * W * (h + 1)]
        acc = None
        for p in range(rp + 1):
            Gt = ((srow == d2) & (prow == p)).astype(jnp.float32)
            gath = jax.lax.dot_general(sbh, Gt, (((1,), (1,)), ((), ())),
                                       preferred_element_type=jnp.float32)
            acc = gath if acc is None else jnp.maximum(acc, gath)
        accs.append(acc)
    o_ref[0] = jnp.concatenate(accs, axis=1)


def kernel(x, source, W_group, r):
    B, N, D = x.shape
    dq = W_group.shape[0]
    rp = min(D // dq, W // 2)
    ns = W - rp
    num_windows = N // W
    tw = B * num_windows

    x2 = x.reshape(B * N, D)

    xm2, scol, pcol = pl.pallas_call(
        functools.partial(_stage1_body, rp=rp, dq=dq),
        grid=(tw // NW1,),
        in_specs=[
            pl.BlockSpec((NW1 * W, D), lambda i: (i, 0)),
            pl.BlockSpec((dq, D), lambda i: (0, 0)),
        ],
        out_specs=[
            pl.BlockSpec((NW1 * ns, D), lambda i: (i, 0)),
            pl.BlockSpec((NW1 // 8, 1, 8 * W), lambda i: (i, 0, 0)),
            pl.BlockSpec((NW1 // 8, 1, 8 * W), lambda i: (i, 0, 0)),
        ],
        out_shape=[
            jax.ShapeDtypeStruct((tw * ns, D), jnp.float32),
            jax.ShapeDtypeStruct((tw // 8, 1, 8 * W), jnp.int32),
            jax.ShapeDtypeStruct((tw // 8, 1, 8 * W), jnp.int32),
        ],
    )(x2, W_group)
    x_merged = xm2.reshape(B, num_windows * ns, D)

    n_rows = source.shape[1]
    s_m = pl.pallas_call(
        functools.partial(_stage3_body, rp=rp, ns=ns),
        grid=(B, num_windows // WB3),
        in_specs=[
            pl.BlockSpec((WB3 // 8, 1, 8 * W),
                         lambda b, wb: (b * (num_windows // WB3) + wb, 0, 0)),
            pl.BlockSpec((WB3 // 8, 1, 8 * W),
                         lambda b, wb: (b * (num_windows // WB3) + wb, 0, 0)),
            pl.BlockSpec((1, n_rows, WB3 * W), lambda b, wb: (b, 0, wb)),
        ],
        out_specs=pl.BlockSpec((1, n_rows, WB3 * ns), lambda b, wb: (b, 0, wb)),
        out_shape=jax.ShapeDtypeStruct((B, n_rows, num_windows * ns), jnp.float32),
    )(scol, pcol, source)

    return (x_merged, s_m)
